# Initial kernel scaffold; baseline (speedup 1.0000x reference)
import jax
import jax.numpy as jnp
from jax.experimental import pallas as pl


def kernel(x1, u1, x2, u2, batch1, batch2, params):
    raise NotImplementedError("write your pallas kernel here")


# speedup vs baseline: 4.2710x; 4.2710x over previous
import jax
import jax.numpy as jnp
from jax.experimental import pallas as pl
from jax.experimental.pallas import tpu as pltpu

_B = 256
_F = 128


def _mm(a, b):
    return jax.lax.dot_general(a, b, (((1,), (0,)), ((), ())),
                               preferred_element_type=jnp.float32)


def _relu(x):
    return jnp.maximum(x, 0.0)


def _make_pass1(R, grid):
    def body(bc1, br1, bc2, br2, x1, x2, u1, u2,
             A1, Bu1, b11, A2, Bu2, b12,
             s1o, s2o, c1o, c2o, t1, t2):
        step = pl.program_id(0)

        @pl.when(step == 0)
        def _init():
            t1[...] = _mm(u1[...], Bu1[...]) + b11[...]
            t2[...] = _mm(u2[...], Bu2[...]) + b12[...]
            s1o[...] = jnp.zeros((_B, _F), jnp.float32)
            s2o[...] = jnp.zeros((_B, _F), jnp.float32)
            c1o[...] = jnp.zeros((_B, 8), jnp.float32)
            c2o[...] = jnp.zeros((_B, 8), jnp.float32)

        iota_l = jax.lax.broadcasted_iota(jnp.int32, (R, _B), 1)
        iota_s = jax.lax.broadcasted_iota(jnp.int32, (_B, R), 0)
        ones8 = jnp.ones((R, 8), jnp.float32)
        for bc, br, x, a, t, so, co in ((bc1, br1, x1, A1, t1, s1o, c1o),
                                        (bc2, br2, x2, A2, t2, s2o, c2o)):
            oh = (bc[...] == iota_l).astype(jnp.float32)
            oht = (br[0] == iota_s).astype(jnp.float32)
            h = _relu(_mm(x[...], a[...]) + _mm(oh, t[...]))
            so[...] += _mm(oht, h)
            co[...] += _mm(oht, ones8)
    return body


def _make_pass2(R, grid):
    def body(bc1, br1, bc2, br2, x1, x2, u1, u2, s11, s12, cn1, cn2,
             A1, Bu1, b11, W21, b21, Va1, Vu1, d11, V21, d21,
             A2, Bu2, b12, W22, b22, Va2, Vu2, d12, V22, d22,
             Wfa, Wfb, bf1, Wf2, bf2,
             out,
             tc11, tc21, G1, un1, a1, tc12, tc22, G2, un2, a2):
        step = pl.program_id(0)

        @pl.when(step == 0)
        def _init():
            for (s1, cn, u, a, bu, b1, w2, b2, va, vu, d1, v2, d2,
                 tc1, tc2, g, un, acc) in (
                    (s11, cn1, u1, A1, Bu1, b11, W21, b21, Va1, Vu1, d11,
                     V21, d21, tc11, tc21, G1, un1, a1),
                    (s12, cn2, u2, A2, Bu2, b12, W22, b22, Va2, Vu2, d12,
                     V22, d22, tc12, tc22, G2, un2, a2)):
                agg1 = _mm(s1[...], w2[...]) + cn[:, 0:1] * b2[...]
                zz = _relu(_mm(agg1, va[...]) + _mm(u[...], vu[...]) + d1[...])
                unew = _mm(zz, v2[...]) + d2[...]
                un[...] = unew
                tc1[...] = _mm(u[...], bu[...]) + b1[...]
                g[...] = _mm(w2[...], a[...])
                e = _mm(b2[...], a[...])
                tc2[...] = _mm(unew, bu[...]) + b1[...] + e
                acc[...] = jnp.zeros((_B, _F), jnp.float32)
            zf = _relu(_mm(un1[...], Wfa[...]) + _mm(un2[...], Wfb[...])
                       + bf1[...])
            out[0, :, :] = _mm(zf, Wf2[...]) + bf2[...]

        iota_l = jax.lax.broadcasted_iota(jnp.int32, (R, _B), 1)
        iota_s = jax.lax.broadcasted_iota(jnp.int32, (_B, R), 0)
        for bc, br, x, a, tc1, tc2, g, acc in (
                (bc1, br1, x1, A1, tc11, tc21, G1, a1),
                (bc2, br2, x2, A2, tc12, tc22, G2, a2)):
            oh = (bc[...] == iota_l).astype(jnp.float32)
            oht = (br[0] == iota_s).astype(jnp.float32)
            h = _relu(_mm(x[...], a[...]) + _mm(oh, tc1[...]))
            h2 = _relu(_mm(h, g[...]) + _mm(oh, tc2[...]))
            acc[...] += _mm(oht, h2)

        @pl.when(step == grid - 1)
        def _fini():
            u2s = []
            for cn, w2, b2, va, vu, d1, v2, d2, un, acc in (
                    (cn1, W21, b21, Va1, Vu1, d11, V21, d21, un1, a1),
                    (cn2, W22, b22, Va2, Vu2, d12, V22, d22, un2, a2)):
                agg2 = _mm(acc[...], w2[...]) + cn[:, 0:1] * b2[...]
                zz = _relu(_mm(agg2, va[...]) + _mm(un[...], vu[...])
                           + d1[...])
                u2s.append(_mm(zz, v2[...]) + d2[...])
            zf = _relu(_mm(u2s[0], Wfa[...]) + _mm(u2s[1], Wfb[...])
                       + bf1[...])
            out[1, :, :] = _mm(zf, Wf2[...]) + bf2[...]
    return body


def kernel(x1, u1, x2, u2, batch1, batch2, params):
    n = x1.shape[0]
    R = 2000 if n % 2000 == 0 else 8
    assert n % R == 0
    grid = n // R

    (W11, b11), (W21, b21) = params['gnn1_node']
    (W12, b12), (W22, b22) = params['gnn2_node']
    (Vg11, d11), (Vg21, d21) = params['gnn1_glob']
    (Vg12, d12), (Vg22, d22) = params['gnn2_glob']
    (Wf1, bf1), (Wf2, bf2) = params['final']

    A1, Bu1 = W11[:_F], W11[_F:]
    A2, Bu2 = W12[:_F], W12[_F:]
    Va1, Vu1 = Vg11[:_F], Vg11[_F:]
    Va2, Vu2 = Vg12[:_F], Vg12[_F:]
    Wfa, Wfb = Wf1[:_F], Wf1[_F:]

    row = lambda v: v.reshape(1, -1)
    bc1 = batch1.reshape(n, 1)
    br1 = batch1.reshape(grid, 1, R)
    bc2 = batch2.reshape(n, 1)
    br2 = batch2.reshape(grid, 1, R)

    bc_spec = pl.BlockSpec((R, 1), lambda i: (i, 0))
    br_spec = pl.BlockSpec((1, 1, R), lambda i: (i, 0, 0))
    x_spec = pl.BlockSpec((R, _F), lambda i: (i, 0))
    full = lambda arr: pl.BlockSpec(arr.shape, lambda i: (0,) * arr.ndim)
    acc_spec = pl.BlockSpec((_B, _F), lambda i: (0, 0))
    cnt_spec = pl.BlockSpec((_B, 8), lambda i: (0, 0))
    f32 = jnp.float32

    p1_weights = (A1, Bu1, row(b11), A2, Bu2, row(b12))
    s11, s12, cn1, cn2 = pl.pallas_call(
        _make_pass1(R, grid),
        grid=(grid,),
        in_specs=[bc_spec, br_spec, bc_spec, br_spec, x_spec, x_spec,
                  full(u1), full(u2)] + [full(w) for w in p1_weights],
        out_specs=[acc_spec, acc_spec, cnt_spec, cnt_spec],
        out_shape=[jax.ShapeDtypeStruct((_B, _F), f32),
                   jax.ShapeDtypeStruct((_B, _F), f32),
                   jax.ShapeDtypeStruct((_B, 8), f32),
                   jax.ShapeDtypeStruct((_B, 8), f32)],
        scratch_shapes=[pltpu.VMEM((_B, _F), f32)] * 2,
        compiler_params=pltpu.CompilerParams(
            dimension_semantics=("arbitrary",)),
    )(bc1, br1, bc2, br2, x1, x2, u1, u2, *p1_weights)

    p2_weights = (A1, Bu1, row(b11), W21, row(b21), Va1, Vu1, row(d11),
                  Vg21, row(d21),
                  A2, Bu2, row(b12), W22, row(b22), Va2, Vu2, row(d12),
                  Vg22, row(d22),
                  Wfa, Wfb, row(bf1), Wf2, row(bf2))
    out = pl.pallas_call(
        _make_pass2(R, grid),
        grid=(grid,),
        in_specs=[bc_spec, br_spec, bc_spec, br_spec, x_spec, x_spec,
                  full(u1), full(u2), acc_spec, acc_spec, cnt_spec,
                  cnt_spec] + [full(w) for w in p2_weights],
        out_specs=pl.BlockSpec((2, _B, 2), lambda i: (0, 0, 0)),
        out_shape=jax.ShapeDtypeStruct((2, _B, 2), f32),
        scratch_shapes=[pltpu.VMEM((_B, _F), f32), pltpu.VMEM((_B, _F), f32),
                        pltpu.VMEM((_F, _F), f32), pltpu.VMEM((_B, _F), f32),
                        pltpu.VMEM((_B, _F), f32)] * 2,
        compiler_params=pltpu.CompilerParams(
            dimension_semantics=("arbitrary",)),
    )(bc1, br1, bc2, br2, x1, x2, u1, u2, s11, s12, cn1, cn2, *p2_weights)
    return out
